# baseline (speedup 1.0000x reference)
import jax
import jax.numpy as jnp
from jax.experimental import pallas as pl
from jax.experimental.pallas import tpu as pltpu

_N = 10000
_BM = 400
_NCHUNK = 10
_CH = _BM // _NCHUNK
_NSLOT = 3


def _start_stripe(adj_ref, abuf, sems, slot, step):
    for c in range(_NCHUNK):
        pltpu.make_async_copy(
            adj_ref.at[pl.ds(step * _BM + c * _CH, _CH), :],
            abuf.at[slot, pl.ds(c * _CH, _CH), :],
            sems.at[slot, c],
        ).start()


def _wait_stripe(adj_ref, abuf, sems, slot, step):
    for c in range(_NCHUNK):
        pltpu.make_async_copy(
            adj_ref.at[pl.ds(step * _BM + c * _CH, _CH), :],
            abuf.at[slot, pl.ds(c * _CH, _CH), :],
            sems.at[slot, c],
        ).wait()


def _gc_kernel(x_ref, adj_ref, w_ref, b_ref, out_ref, sup_ref, abuf, sems):
    i = pl.program_id(0)
    g = pl.num_programs(0)
    slot = jax.lax.rem(i, _NSLOT)

    @pl.when(i == 0)
    def _():
        _start_stripe(adj_ref, abuf, sems, 0, 0)
        _start_stripe(adj_ref, abuf, sems, 1, 1)
        sup_ref[...] = jnp.dot(
            x_ref[...], w_ref[...], preferred_element_type=jnp.float32
        )

    @pl.when(i + 2 < g)
    def _():
        _start_stripe(adj_ref, abuf, sems, jax.lax.rem(i + 2, _NSLOT), i + 2)

    _wait_stripe(adj_ref, abuf, sems, slot, i)
    out_ref[...] = (
        jnp.dot(abuf[slot], sup_ref[...], preferred_element_type=jnp.float32)
        + b_ref[...]
    )


def kernel(input, adj, W, b):
    n, d_in = input.shape
    d_out = W.shape[1]
    b2 = b.reshape(1, d_out)
    return pl.pallas_call(
        _gc_kernel,
        grid=(n // _BM,),
        in_specs=[
            pl.BlockSpec((n, d_in), lambda i: (0, 0)),
            pl.BlockSpec(memory_space=pltpu.MemorySpace.HBM),
            pl.BlockSpec((d_in, d_out), lambda i: (0, 0)),
            pl.BlockSpec((1, d_out), lambda i: (0, 0)),
        ],
        out_specs=pl.BlockSpec((_BM, d_out), lambda i: (i, 0)),
        out_shape=jax.ShapeDtypeStruct((n, d_out), jnp.float32),
        scratch_shapes=[
            pltpu.VMEM((n, d_out), jnp.float32),
            pltpu.VMEM((_NSLOT, _BM, n), jnp.float32),
            pltpu.SemaphoreType.DMA((_NSLOT, _NCHUNK)),
        ],
    )(input, adj, W, b2)


# speedup vs baseline: 1.0363x; 1.0363x over previous
import jax
import jax.numpy as jnp
from jax.experimental import pallas as pl
from jax.experimental.pallas import tpu as pltpu

_BM = 400


def _gc_kernel(x_ref, adj_ref, w_ref, b_ref, out_ref, sup_ref):
    @pl.when(pl.program_id(0) == 0)
    def _():
        sup_ref[...] = jnp.dot(
            x_ref[...], w_ref[...], preferred_element_type=jnp.float32
        )

    out_ref[...] = (
        jnp.dot(adj_ref[...], sup_ref[...], preferred_element_type=jnp.float32)
        + b_ref[...]
    )


def kernel(input, adj, W, b):
    n, d_in = input.shape
    d_out = W.shape[1]
    b2 = b.reshape(1, d_out)
    return pl.pallas_call(
        _gc_kernel,
        grid=(n // _BM,),
        in_specs=[
            pl.BlockSpec((n, d_in), lambda i: (0, 0)),
            pl.BlockSpec((_BM, n), lambda i: (i, 0)),
            pl.BlockSpec((d_in, d_out), lambda i: (0, 0)),
            pl.BlockSpec((1, d_out), lambda i: (0, 0)),
        ],
        out_specs=pl.BlockSpec((_BM, d_out), lambda i: (i, 0)),
        out_shape=jax.ShapeDtypeStruct((n, d_out), jnp.float32),
        scratch_shapes=[pltpu.VMEM((n, d_out), jnp.float32)],
    )(input, adj, W, b2)


# speedup vs baseline: 1.0370x; 1.0007x over previous
import jax
import jax.numpy as jnp
from jax.experimental import pallas as pl
from jax.experimental.pallas import tpu as pltpu

_BM = 400


def _gc_kernel(x_ref, adj_ref, w_ref, b_ref, out_ref, sup_ref):
    @pl.when(pl.program_id(0) == 0)
    def _():
        sup_ref[...] = jnp.dot(
            x_ref[...], w_ref[...], preferred_element_type=jnp.float32
        )

    out_ref[...] = (
        jnp.dot(
            adj_ref[...].astype(jnp.bfloat16),
            sup_ref[...].astype(jnp.bfloat16),
            preferred_element_type=jnp.float32,
        )
        + b_ref[...]
    )


def kernel(input, adj, W, b):
    n, d_in = input.shape
    d_out = W.shape[1]
    b2 = b.reshape(1, d_out)
    return pl.pallas_call(
        _gc_kernel,
        grid=(n // _BM,),
        in_specs=[
            pl.BlockSpec((n, d_in), lambda i: (0, 0)),
            pl.BlockSpec((_BM, n), lambda i: (i, 0)),
            pl.BlockSpec((d_in, d_out), lambda i: (0, 0)),
            pl.BlockSpec((1, d_out), lambda i: (0, 0)),
        ],
        out_specs=pl.BlockSpec((_BM, d_out), lambda i: (i, 0)),
        out_shape=jax.ShapeDtypeStruct((n, d_out), jnp.float32),
        scratch_shapes=[pltpu.VMEM((n, d_out), jnp.float32)],
    )(input, adj, W, b2)
